# baseline (speedup 1.0000x reference)
import jax
import jax.numpy as jnp
from jax.experimental import pallas as pl
from jax.experimental.pallas import tpu as pltpu

B = 4
T = 2048
D_PROJ = 1024
D_C = 512
D_MODEL = 1024
N_HEADS = 16
HD = D_MODEL // N_HEADS
S0 = 64
GATE_R = 8
PER_SPK = 32
SQ = 2 * PER_SPK
SIGMA = 4.0

NCH = 4

_DNT = (((1,), (1,)), ((), ()))
F32 = jnp.float32
BF16 = jnp.bfloat16


def _dott(a, b):
    return jax.lax.dot_general(a, b, _DNT, preferred_element_type=F32)


def _dot(a, b):
    return jax.lax.dot_general(a, b, (((1,), (0,)), ((), ())),
                               preferred_element_type=F32)


def _spk_track(h_ref, a_ref, sp_ref, wkv_ref, bkv_ref, wq, bq, wqin, bqin):
    a_row = a_ref[0]
    s_row = sp_ref[0]
    s_col = jnp.transpose(s_row)

    t_row = jax.lax.broadcasted_iota(jnp.int32, (S0, T), 1)
    dist = t_row - s_col

    acc = jnp.zeros((S0, 1), F32)
    cnt = jnp.zeros((S0, 1), jnp.int32)
    for off in range(-GATE_R, GATE_R + 1):
        m = dist == off
        tap = jnp.sum(jnp.where(m, a_row, 0.0), axis=1, keepdims=True)
        acc = acc + tap
        idx = s_col + off
        cnt = cnt + ((idx >= 0) & (idx < T)).astype(jnp.int32)
    scores = acc / jnp.maximum(cnt, 1).astype(F32)
    scores_row = jnp.transpose(scores)

    ii = jax.lax.broadcasted_iota(jnp.int32, (S0, S0), 0)
    jj = jax.lax.broadcasted_iota(jnp.int32, (S0, S0), 1)
    gt = (scores_row > scores).astype(jnp.int32)
    eq = ((scores_row == scores) & (jj < ii)).astype(jnp.int32)
    rank = jnp.sum(gt + eq, axis=1, keepdims=True)
    rank_row = jnp.transpose(rank)

    r_col = jax.lax.broadcasted_iota(jnp.int32, (PER_SPK, 1), 0)
    sel = (rank_row == r_col).astype(jnp.int32)
    p = jnp.sum(sel * s_row, axis=1, keepdims=True)
    conf = jnp.sum(sel.astype(F32) * scores_row, axis=1, keepdims=True)
    gate = jax.nn.sigmoid(2.0 * conf)

    t2 = jax.lax.broadcasted_iota(jnp.int32, (PER_SPK, T), 1)
    d2 = t2 - p
    win = (d2 >= -GATE_R) & (d2 <= GATE_R)
    df = d2.astype(F32) * (1.0 / SIGMA)
    w = jnp.where(win, jnp.exp(-0.5 * df * df) * a_row, 0.0)
    wsum = jnp.sum(w, axis=1, keepdims=True)
    wn = (w / (wsum + 1e-6)).astype(BF16)
    z = _dot(wn, h_ref[0].astype(BF16)).astype(BF16)

    k_seed = _dott(z, wkv_ref[0]) + bkv_ref[...]
    qk = jnp.tanh(_dott(k_seed.astype(BF16), wq) + bq).astype(BF16)
    return (_dott(qk, wqin) + bqin).astype(BF16), gate


def _mega_kernel(h0_ref, a0_ref, sp0_ref, wkv0_ref, bkv0_ref,
                 h1_ref, a1_ref, sp1_ref, wkv1_ref, bkv1_ref,
                 wq_ref, bq_ref, wqi_ref, bqi_ref,
                 wki_ref, wvi_ref, wmem_ref, bmem_ref, bvi_ref,
                 pf_ref, opw_ref, opb_ref, wo_ref, bo_ref,
                 a0s_ref, a1s_ref, tags_ref,
                 out_ref,
                 g_s, wkf_s, wvf_s, bvf_s, o_s, qp_s, m_s, l_s, acc_s):
    b = pl.program_id(0)
    ph = pl.program_id(1)

    @pl.when((ph == 0) & (b == 0))
    def _():
        wmem = wmem_ref[...]
        wkf_s[...] = _dot(wki_ref[...], wmem).astype(BF16)
        wvf_s[...] = _dot(wvi_ref[...], wmem).astype(BF16)
        bvf_s[...] = (_dott(bmem_ref[...].astype(BF16), wvi_ref[...])
                      + bvi_ref[...])

    @pl.when(ph == 0)
    def _():
        wq = wq_ref[...]
        bq = bq_ref[...]
        wqi = wqi_ref[...]
        bqi = bqi_ref[...]
        q0, g0 = _spk_track(h0_ref, a0_ref, sp0_ref, wkv0_ref, bkv0_ref,
                            wq, bq, wqi, bqi)
        q1, g1 = _spk_track(h1_ref, a1_ref, sp1_ref, wkv1_ref, bkv1_ref,
                            wq, bq, wqi, bqi)
        g_s[0:PER_SPK] = g0
        g_s[PER_SPK:SQ] = g1
        qa = jnp.concatenate([q0, q1], axis=0)
        for h in range(N_HEADS):
            qp = _dot(qa[:, h * HD:(h + 1) * HD],
                      wkf_s[h * HD:(h + 1) * HD, :])
            qp_s[h * SQ:(h + 1) * SQ] = qp.astype(BF16)

    @pl.when(ph >= 1)
    def _():
        x = pf_ref[0].astype(BF16)
        sc = _dott(qp_s[...], x) * (1.0 / (HD ** 0.5))
        m_loc = jnp.max(sc, axis=1, keepdims=True)
        m_old = jnp.where(ph == 1,
                          jnp.full((N_HEADS * SQ, 1), -1e30, F32), m_s[...])
        m_new = jnp.maximum(m_old, m_loc)
        resc = jnp.exp(m_old - m_new)
        m_s[...] = m_new
        e = jnp.exp(sc - m_new)
        lloc = jnp.sum(e, axis=1, keepdims=True)
        l_old = jnp.where(ph == 1, jnp.zeros((N_HEADS * SQ, 1), F32),
                          l_s[...])
        l_s[...] = l_old * resc + lloc
        u_part = _dot(e.astype(BF16), x)
        acc_old = jnp.where(ph == 1,
                            jnp.zeros((N_HEADS * SQ, D_PROJ), F32),
                            acc_s[...])
        acc_s[...] = acc_old * resc + u_part

    @pl.when(ph == NCH)
    def _():
        u = (acc_s[...] / l_s[...]).astype(BF16)
        bvf = bvf_s[...]
        for h in range(N_HEADS):
            oh = _dott(u[h * SQ:(h + 1) * SQ],
                       wvf_s[h * HD:(h + 1) * HD, :])
            o_s[:, h * HD:(h + 1) * HD] = oh + bvf[:, h * HD:(h + 1) * HD]

        o = o_s[...].astype(BF16)
        f = _dott(o, opw_ref[...]) + opb_ref[...]
        f = _dott(f.astype(BF16), wo_ref[...]) + bo_ref[...]
        g = g_s[...]
        a0 = a0s_ref[0, :, 0:1]
        a1 = a1s_ref[0, :, 0:1]
        den = a0 + a1 + 1e-6
        tags = tags_ref[...]
        slot = (a0 / den) * tags[0:1, :] + (a1 / den) * tags[1:2, :]
        out_ref[0] = f * g + slot


def kernel(proj_feats, h_ctc_0, h_ctc_1, A_0, A_1, spikes_0, spikes_1,
           W_mem, b_mem, W_kv_0, b_kv_0, W_kv_1, b_kv_1, W_q, b_q, W_o, b_o,
           in_proj_w, in_proj_b, out_proj_w, out_proj_b, tags):
    wqi = in_proj_w[0:D_MODEL].astype(BF16)
    wki = in_proj_w[D_MODEL:2 * D_MODEL].astype(BF16)
    wvi = in_proj_w[2 * D_MODEL:3 * D_MODEL].astype(BF16)
    bqi = in_proj_b[0:D_MODEL].reshape(1, D_MODEL)
    bvi = in_proj_b[2 * D_MODEL:3 * D_MODEL].reshape(1, D_MODEL)

    bspec3 = lambda shape: pl.BlockSpec(shape, lambda b, p: (b, 0, 0))
    cspec = lambda shape: pl.BlockSpec(
        shape, lambda b, p, _n=len(shape): tuple(0 for _ in range(_n)))

    in_specs = [
        bspec3((1, T, D_C)), bspec3((1, 1, T)), bspec3((1, 1, S0)),
        cspec((1, D_MODEL, D_C)), cspec((1, D_MODEL)),
        bspec3((1, T, D_C)), bspec3((1, 1, T)), bspec3((1, 1, S0)),
        cspec((1, D_MODEL, D_C)), cspec((1, D_MODEL)),
        cspec((D_MODEL, D_MODEL)), cspec((1, D_MODEL)),
        cspec((D_MODEL, D_MODEL)), cspec((1, D_MODEL)),
        cspec((D_MODEL, D_MODEL)), cspec((D_MODEL, D_MODEL)),
        cspec((D_MODEL, D_PROJ)), cspec((1, D_MODEL)), cspec((1, D_MODEL)),
        pl.BlockSpec((1, T // NCH, D_PROJ),
                     lambda b, p: (b, jax.lax.max(p - 1, 0), 0)),
        cspec((D_MODEL, D_MODEL)), cspec((1, D_MODEL)),
        cspec((D_MODEL, D_MODEL)), cspec((1, D_MODEL)),
        bspec3((1, SQ, T // SQ)), bspec3((1, SQ, T // SQ)),
        cspec((2, D_MODEL)),
    ]

    out = pl.pallas_call(
        _mega_kernel,
        grid=(B, NCH + 1),
        in_specs=in_specs,
        out_specs=pl.BlockSpec((1, SQ, D_MODEL), lambda b, p: (b, 0, 0)),
        out_shape=jax.ShapeDtypeStruct((B, SQ, D_MODEL), F32),
        scratch_shapes=[
            pltpu.VMEM((SQ, 1), F32),
            pltpu.VMEM((D_MODEL, D_PROJ), BF16),
            pltpu.VMEM((D_MODEL, D_PROJ), BF16),
            pltpu.VMEM((1, D_MODEL), F32),
            pltpu.VMEM((SQ, D_MODEL), F32),
            pltpu.VMEM((N_HEADS * SQ, D_PROJ), BF16),
            pltpu.VMEM((N_HEADS * SQ, 1), F32),
            pltpu.VMEM((N_HEADS * SQ, 1), F32),
            pltpu.VMEM((N_HEADS * SQ, D_PROJ), F32),
        ],
    )(h_ctc_0, A_0.reshape(B, 1, T), spikes_0.reshape(B, 1, S0),
      W_kv_0[:D_MODEL].reshape(1, D_MODEL, D_C).astype(BF16),
      b_kv_0[:D_MODEL].reshape(1, D_MODEL),
      h_ctc_1, A_1.reshape(B, 1, T), spikes_1.reshape(B, 1, S0),
      W_kv_1[:D_MODEL].reshape(1, D_MODEL, D_C).astype(BF16),
      b_kv_1[:D_MODEL].reshape(1, D_MODEL),
      W_q.astype(BF16), b_q.reshape(1, D_MODEL), wqi, bqi,
      wki, wvi, W_mem.astype(BF16), b_mem.reshape(1, D_MODEL), bvi,
      proj_feats, out_proj_w.astype(BF16),
      out_proj_b.reshape(1, D_MODEL), W_o.astype(BF16),
      b_o.reshape(1, D_MODEL),
      A_0.reshape(B, SQ, T // SQ), A_1.reshape(B, SQ, T // SQ), tags)
    return out


# speedup vs baseline: 1.1391x; 1.1391x over previous
import jax
import jax.numpy as jnp
from jax.experimental import pallas as pl
from jax.experimental.pallas import tpu as pltpu

B = 4
T = 2048
D_PROJ = 1024
D_C = 512
D_MODEL = 1024
N_HEADS = 16
HD = D_MODEL // N_HEADS
S0 = 64
GATE_R = 8
PER_SPK = 32
SQ = 2 * PER_SPK
SIGMA = 4.0

NCH = 4

_DNT = (((1,), (1,)), ((), ()))
F32 = jnp.float32
BF16 = jnp.bfloat16


def _dott(a, b):
    return jax.lax.dot_general(a, b, _DNT, preferred_element_type=F32)


def _dot(a, b):
    return jax.lax.dot_general(a, b, (((1,), (0,)), ((), ())),
                               preferred_element_type=F32)


def _spk_track(h_ref, a_ref, sp_ref, wkv_ref, bkv_ref, wq, bq, wqin, bqin):
    a_row = a_ref[0]
    s_row = sp_ref[0]
    s_col = jnp.transpose(s_row)

    t_row = jax.lax.broadcasted_iota(jnp.int32, (S0, T), 1)
    dist = t_row - s_col

    acc = jnp.zeros((S0, 1), F32)
    cnt = jnp.zeros((S0, 1), jnp.int32)
    for off in range(-GATE_R, GATE_R + 1):
        m = dist == off
        tap = jnp.sum(jnp.where(m, a_row, 0.0), axis=1, keepdims=True)
        acc = acc + tap
        idx = s_col + off
        cnt = cnt + ((idx >= 0) & (idx < T)).astype(jnp.int32)
    scores = acc / jnp.maximum(cnt, 1).astype(F32)
    scores_row = jnp.transpose(scores)

    ii = jax.lax.broadcasted_iota(jnp.int32, (S0, S0), 0)
    jj = jax.lax.broadcasted_iota(jnp.int32, (S0, S0), 1)
    gt = (scores_row > scores).astype(jnp.int32)
    eq = ((scores_row == scores) & (jj < ii)).astype(jnp.int32)
    rank = jnp.sum(gt + eq, axis=1, keepdims=True)
    rank_row = jnp.transpose(rank)

    r_col = jax.lax.broadcasted_iota(jnp.int32, (PER_SPK, 1), 0)
    sel = (rank_row == r_col).astype(jnp.int32)
    p = jnp.sum(sel * s_row, axis=1, keepdims=True)
    conf = jnp.sum(sel.astype(F32) * scores_row, axis=1, keepdims=True)
    gate = jax.nn.sigmoid(2.0 * conf)

    t2 = jax.lax.broadcasted_iota(jnp.int32, (PER_SPK, T), 1)
    d2 = t2 - p
    win = (d2 >= -GATE_R) & (d2 <= GATE_R)
    df = d2.astype(F32) * (1.0 / SIGMA)
    w = jnp.where(win, jnp.exp(-0.5 * df * df) * a_row, 0.0)
    wsum = jnp.sum(w, axis=1, keepdims=True)
    wn = (w / (wsum + 1e-6)).astype(BF16)
    z = _dot(wn, h_ref[0].astype(BF16)).astype(BF16)

    k_seed = _dott(z, wkv_ref[0]) + bkv_ref[...]
    qk = jnp.tanh(_dott(k_seed.astype(BF16), wq) + bq).astype(BF16)
    return (_dott(qk, wqin) + bqin).astype(BF16), gate


def _prep_kernel(h0_ref, a0_ref, sp0_ref, wkv0_ref, bkv0_ref,
                 h1_ref, a1_ref, sp1_ref, wkv1_ref, bkv1_ref,
                 wq_ref, bq_ref, wqi_ref, bqi_ref,
                 wki_ref, wvi_ref, wmem_ref, bmem_ref, bvi_ref,
                 qp_out, g_out, wvf_out, bvf_out, wkf_s):
    b = pl.program_id(0)

    @pl.when(b == 0)
    def _():
        wmem = wmem_ref[...]
        wkf_s[...] = _dot(wki_ref[...], wmem).astype(BF16)
        wvf_out[...] = _dot(wvi_ref[...], wmem).astype(BF16)
        bvf_out[...] = (_dott(bmem_ref[...].astype(BF16), wvi_ref[...])
                        + bvi_ref[...])

    q0, g0 = _spk_track(h0_ref, a0_ref, sp0_ref, wkv0_ref, bkv0_ref,
                        wq_ref[...], bq_ref[...], wqi_ref[...], bqi_ref[...])
    q1, g1 = _spk_track(h1_ref, a1_ref, sp1_ref, wkv1_ref, bkv1_ref,
                        wq_ref[...], bq_ref[...], wqi_ref[...], bqi_ref[...])
    g_out[0, 0:PER_SPK] = g0
    g_out[0, PER_SPK:SQ] = g1
    qa = jnp.concatenate([q0, q1], axis=0)
    for h in range(N_HEADS):
        qp = _dot(qa[:, h * HD:(h + 1) * HD], wkf_s[h * HD:(h + 1) * HD, :])
        qp_out[0, h * SQ:(h + 1) * SQ] = qp.astype(BF16)


def _attn_kernel(pf_ref, qp_ref, wvf_ref, bvf_ref, opw_ref, opb_ref,
                 wo_ref, bo_ref, g_ref, a0s_ref, a1s_ref, tags_ref,
                 out_ref, o_s):
    x = pf_ref[0].astype(BF16)
    sc = _dott(qp_ref[0], x) * (1.0 / (HD ** 0.5))
    m = jnp.max(sc, axis=1, keepdims=True)
    e = jnp.exp(sc - m)
    l = jnp.sum(e, axis=1, keepdims=True)
    u = (_dot(e.astype(BF16), x) / l).astype(BF16)
    bvf = bvf_ref[...]
    for h in range(N_HEADS):
        oh = _dott(u[h * SQ:(h + 1) * SQ],
                   wvf_ref[h * HD:(h + 1) * HD, :])
        o_s[:, h * HD:(h + 1) * HD] = oh + bvf[:, h * HD:(h + 1) * HD]

    o = o_s[...].astype(BF16)
    f = _dott(o, opw_ref[...]) + opb_ref[...]
    f = _dott(f.astype(BF16), wo_ref[...]) + bo_ref[...]
    g = g_ref[0]
    a0 = a0s_ref[0, :, 0:1]
    a1 = a1s_ref[0, :, 0:1]
    den = a0 + a1 + 1e-6
    tags = tags_ref[...]
    slot = (a0 / den) * tags[0:1, :] + (a1 / den) * tags[1:2, :]
    out_ref[0] = f * g + slot


def kernel(proj_feats, h_ctc_0, h_ctc_1, A_0, A_1, spikes_0, spikes_1,
           W_mem, b_mem, W_kv_0, b_kv_0, W_kv_1, b_kv_1, W_q, b_q, W_o, b_o,
           in_proj_w, in_proj_b, out_proj_w, out_proj_b, tags):
    wqi = in_proj_w[0:D_MODEL].astype(BF16)
    wki = in_proj_w[D_MODEL:2 * D_MODEL].astype(BF16)
    wvi = in_proj_w[2 * D_MODEL:3 * D_MODEL].astype(BF16)
    bqi = in_proj_b[0:D_MODEL].reshape(1, D_MODEL)
    bvi = in_proj_b[2 * D_MODEL:3 * D_MODEL].reshape(1, D_MODEL)

    bspec3 = lambda shape: pl.BlockSpec(shape, lambda b: (b, 0, 0))
    cspec = lambda shape: pl.BlockSpec(
        shape, lambda b, _n=len(shape): tuple(0 for _ in range(_n)))

    qp, g_all, wvf, bvf = pl.pallas_call(
        _prep_kernel,
        grid=(B,),
        in_specs=[
            bspec3((1, T, D_C)), bspec3((1, 1, T)), bspec3((1, 1, S0)),
            cspec((1, D_MODEL, D_C)), cspec((1, D_MODEL)),
            bspec3((1, T, D_C)), bspec3((1, 1, T)), bspec3((1, 1, S0)),
            cspec((1, D_MODEL, D_C)), cspec((1, D_MODEL)),
            cspec((D_MODEL, D_MODEL)), cspec((1, D_MODEL)),
            cspec((D_MODEL, D_MODEL)), cspec((1, D_MODEL)),
            cspec((D_MODEL, D_MODEL)), cspec((D_MODEL, D_MODEL)),
            cspec((D_MODEL, D_PROJ)), cspec((1, D_MODEL)),
            cspec((1, D_MODEL)),
        ],
        out_specs=[
            bspec3((1, N_HEADS * SQ, D_PROJ)),
            bspec3((1, SQ, 1)),
            cspec((D_MODEL, D_PROJ)),
            cspec((1, D_MODEL)),
        ],
        out_shape=[
            jax.ShapeDtypeStruct((B, N_HEADS * SQ, D_PROJ), BF16),
            jax.ShapeDtypeStruct((B, SQ, 1), F32),
            jax.ShapeDtypeStruct((D_MODEL, D_PROJ), BF16),
            jax.ShapeDtypeStruct((1, D_MODEL), F32),
        ],
        scratch_shapes=[pltpu.VMEM((D_MODEL, D_PROJ), BF16)],
    )(h_ctc_0, A_0.reshape(B, 1, T), spikes_0.reshape(B, 1, S0),
      W_kv_0[:D_MODEL].reshape(1, D_MODEL, D_C).astype(BF16),
      b_kv_0[:D_MODEL].reshape(1, D_MODEL),
      h_ctc_1, A_1.reshape(B, 1, T), spikes_1.reshape(B, 1, S0),
      W_kv_1[:D_MODEL].reshape(1, D_MODEL, D_C).astype(BF16),
      b_kv_1[:D_MODEL].reshape(1, D_MODEL),
      W_q.astype(BF16), b_q.reshape(1, D_MODEL), wqi, bqi,
      wki, wvi, W_mem.astype(BF16), b_mem.reshape(1, D_MODEL), bvi)

    out = pl.pallas_call(
        _attn_kernel,
        grid=(B,),
        in_specs=[
            bspec3((1, T, D_PROJ)),
            bspec3((1, N_HEADS * SQ, D_PROJ)),
            cspec((D_MODEL, D_PROJ)), cspec((1, D_MODEL)),
            cspec((D_MODEL, D_MODEL)), cspec((1, D_MODEL)),
            cspec((D_MODEL, D_MODEL)), cspec((1, D_MODEL)),
            bspec3((1, SQ, 1)),
            bspec3((1, SQ, T // SQ)), bspec3((1, SQ, T // SQ)),
            cspec((2, D_MODEL)),
        ],
        out_specs=pl.BlockSpec((1, SQ, D_MODEL), lambda b: (b, 0, 0)),
        out_shape=jax.ShapeDtypeStruct((B, SQ, D_MODEL), F32),
        scratch_shapes=[pltpu.VMEM((SQ, D_MODEL), F32)],
    )(proj_feats, qp, wvf, bvf, out_proj_w.astype(BF16),
      out_proj_b.reshape(1, D_MODEL), W_o.astype(BF16),
      b_o.reshape(1, D_MODEL), g_all,
      A_0.reshape(B, SQ, T // SQ), A_1.reshape(B, SQ, T // SQ), tags)
    return out
